# Initial kernel scaffold; baseline (speedup 1.0000x reference)
import jax
import jax.numpy as jnp
from jax.experimental import pallas as pl


def kernel(x, tables, W0, W1, W2):
    raise NotImplementedError("write your pallas kernel here")


# speedup vs baseline: 2.6077x; 2.6077x over previous
import functools

import jax
import jax.numpy as jnp
import numpy as np
from jax import lax
from jax.experimental import pallas as pl
from jax.experimental.pallas import tpu as pltpu

N = 131072
N_LEVELS = 16
F = 2
T = 1 << 20
BASE_RES = 16
SCALE = 1.4142135624
LC = N_LEVELS * 8

_P1 = np.uint32(2654435761).astype(np.int32).item()
_P2 = np.uint32(805459861).astype(np.int32).item()

_RES = [int(np.floor(BASE_RES * (SCALE ** l))) for l in range(N_LEVELS)]
_DENSE = [(r + 1) ** 3 <= T for r in _RES]


def _idxw_body(xt_ref, idx_ref, wgt_ref):
    x0 = xt_ref[0:1, :]
    x1 = xt_ref[1:2, :]
    x2 = xt_ref[2:3, :]
    for l in range(N_LEVELS):
        res = _RES[l]
        resf = float(res)
        p0 = []
        frac = []
        for xd in (x0, x1, x2):
            pos = xd * resf
            p0f = jnp.floor(pos)
            frac.append(pos - p0f)
            p0.append(p0f.astype(jnp.int32))
        if _DENSE[l]:
            s1 = res + 1
            s2 = s1 * s1
        for c in range(8):
            offs = ((c >> 2) & 1, (c >> 1) & 1, c & 1)
            cd = [jnp.clip(p0[d] + offs[d], 0, res) for d in range(3)]
            if _DENSE[l]:
                flat = cd[0] + cd[1] * s1 + cd[2] * s2
            else:
                flat = (cd[0] ^ (cd[1] * _P1) ^ (cd[2] * _P2)) & (T - 1)
            w = frac[0] if offs[0] == 1 else (1.0 - frac[0])
            for d in (1, 2):
                w = w * (frac[d] if offs[d] == 1 else (1.0 - frac[d]))
            row = l * 8 + c
            idx_ref[row:row + 1, :] = flat + l * T
            wgt_ref[row:row + 1, :] = w


def _compute_idx_wgt(xT, block):
    grid = N // block
    return pl.pallas_call(
        _idxw_body,
        grid=(grid,),
        in_specs=[pl.BlockSpec((3, block), lambda i: (0, i))],
        out_specs=[
            pl.BlockSpec((LC, block), lambda i: (0, i)),
            pl.BlockSpec((LC, block), lambda i: (0, i)),
        ],
        out_shape=[
            jax.ShapeDtypeStruct((LC, N), jnp.int32),
            jax.ShapeDtypeStruct((LC, N), jnp.float32),
        ],
    )(xT)


def _mlp_body(vals0_ref, vals1_ref, wgt_ref, w0t_ref, w1t_ref, w2t_ref,
              out_ref):
    enc_rows = []
    for l in range(N_LEVELS):
        for v_ref in (vals0_ref, vals1_ref):
            b = l * 8
            acc = v_ref[b:b + 1, :] * wgt_ref[b:b + 1, :]
            for c in range(1, 8):
                acc = acc + v_ref[b + c:b + c + 1, :] * wgt_ref[b + c:b + c + 1, :]
            enc_rows.append(acc)
    enc = jnp.concatenate(enc_rows, axis=0)
    h = jnp.maximum(
        lax.dot_general(w0t_ref[...], enc, (((1,), (0,)), ((), ())),
                        preferred_element_type=jnp.float32), 0.0)
    h = jnp.maximum(
        lax.dot_general(w1t_ref[...], h, (((1,), (0,)), ((), ())),
                        preferred_element_type=jnp.float32), 0.0)
    o = lax.dot_general(w2t_ref[...], h, (((1,), (0,)), ((), ())),
                        preferred_element_type=jnp.float32)
    out_ref[...] = jnp.log1p(jnp.exp(-jnp.abs(o))) + jnp.maximum(o, 0.0)


def _mlp(vals0, vals1, wgt, W0, W1, W2, block):
    grid = N // block
    W0T = W0.T
    W1T = W1.T
    W2T = jnp.zeros((8, 128), jnp.float32).at[:F].set(W2.T)
    return pl.pallas_call(
        _mlp_body,
        grid=(grid,),
        in_specs=[
            pl.BlockSpec((LC, block), lambda i: (0, i)),
            pl.BlockSpec((LC, block), lambda i: (0, i)),
            pl.BlockSpec((LC, block), lambda i: (0, i)),
            pl.BlockSpec((128, 32), lambda i: (0, 0)),
            pl.BlockSpec((128, 128), lambda i: (0, 0)),
            pl.BlockSpec((8, 128), lambda i: (0, 0)),
        ],
        out_specs=pl.BlockSpec((8, block), lambda i: (0, i)),
        out_shape=jax.ShapeDtypeStruct((8, N), jnp.float32),
    )(vals0, vals1, wgt, W0T, W1T, W2T)


def kernel(x, tables, W0, W1, W2):
    xT = x.T
    idx, wgt = _compute_idx_wgt(xT, block=2048)
    tf = tables.reshape(N_LEVELS * T, F)
    vals = jnp.take(tf, idx.reshape(-1), axis=0)
    vals0 = vals[:, 0].reshape(LC, N)
    vals1 = vals[:, 1].reshape(LC, N)
    out = _mlp(vals0, vals1, wgt, W0, W1, W2, block=2048)
    return (out[0], out[1])
